# baseline (speedup 1.0000x reference)
import functools

import jax
import jax.numpy as jnp
from jax import lax
from jax.experimental import pallas as pl
from jax.experimental.pallas import tpu as pltpu
from jax.experimental.pallas import tpu_sc as plsc

N, K, D = 10000, 32, 128
NPAD = 10240
NW = 32
NBUF = 2
NB = 4
L = 16
PCH = D // 32

NPW0 = 176
NPW1 = 640 - NPW0
NPW_MAX = max(NPW0, NPW1)

TC_BLK = 512


def _linear_body(x_ref, wt_ref, b_ref, h_ref, hb_ref):
    h = jnp.dot(x_ref[...], wt_ref[...], preferred_element_type=jnp.float32)
    h = h + b_ref[...]
    row = lax.broadcasted_iota(jnp.int32, h.shape, 0) + pl.program_id(0) * TC_BLK
    h = jnp.where(row == 0, jnp.float32(-9e15), h)
    h_ref[...] = h
    hb_ref[...] = h.astype(jnp.bfloat16)


def _linear(x_pad, Wt, b2):
    return pl.pallas_call(
        _linear_body,
        grid=(NPAD // TC_BLK,),
        in_specs=[
            pl.BlockSpec((TC_BLK, D), lambda i: (i, 0)),
            pl.BlockSpec((D, D), lambda i: (0, 0)),
            pl.BlockSpec((1, D), lambda i: (0, 0)),
        ],
        out_specs=[
            pl.BlockSpec((TC_BLK, D), lambda i: (i, 0)),
            pl.BlockSpec((TC_BLK, D), lambda i: (i, 0)),
        ],
        out_shape=[
            jax.ShapeDtypeStruct((NPAD, D), jnp.float32),
            jax.ShapeDtypeStruct((NPAD, D), jnp.bfloat16),
        ],
    )(x_pad, Wt, b2)


def _scale_body(h_ref, w_ref, o_ref):
    o_ref[...] = h_ref[...] * w_ref[:, 0:1]


def _scale(h, w):
    return pl.pallas_call(
        _scale_body,
        grid=(NPAD // TC_BLK,),
        in_specs=[
            pl.BlockSpec((TC_BLK, D), lambda i: (i, 0)),
            pl.BlockSpec((TC_BLK, L), lambda i: (i, 0)),
        ],
        out_specs=pl.BlockSpec((TC_BLK, D), lambda i: (i, 0)),
        out_shape=jax.ShapeDtypeStruct((NPAD, D), jnp.float32),
    )(h, w)


_GATHER_DNUMS = lax.GatherDimensionNumbers(
    offset_dims=(), collapsed_slice_dims=(0,), start_index_map=(0,))


def _perm(x, idx):
    return lax.gather(
        x, idx[:, None], _GATHER_DNUMS, (1,),
        unique_indices=True, indices_are_sorted=False,
        mode=lax.GatherScatterMode.PROMISE_IN_BOUNDS)


def _sc_body(hb_hbm, a2a_hbm, w_hbm, idx_v, hsb_v, w_v, rows_v, sem):
    c = lax.axis_index("c")
    s = lax.axis_index("s")
    base = s * (NPW0 + NPW1) + c * NPW0
    npw = jnp.where(c == 0, NPW0, NPW1)
    ng = npw // NB

    @pl.when(c == 0)
    def _():
        pltpu.sync_copy(a2a_hbm.at[pl.ds(base * K, NPW0 * K)],
                        idx_v.at[pl.ds(0, NPW0 * K)])
        pltpu.sync_copy(hb_hbm.at[pl.ds(base, NPW0)], hsb_v.at[pl.ds(0, NPW0)])

    @pl.when(c == 1)
    def _():
        pltpu.sync_copy(a2a_hbm.at[pl.ds(base * K, NPW1 * K)],
                        idx_v.at[pl.ds(0, NPW1 * K)])
        pltpu.sync_copy(hb_hbm.at[pl.ds(base, NPW1)], hsb_v.at[pl.ds(0, NPW1)])

    lane = lax.iota(jnp.int32, L)

    for b in range(NBUF):
        pltpu.make_async_copy(
            hb_hbm.at[idx_v.at[pl.ds(b * NB * K, NB * K)]], rows_v.at[b], sem
        ).start()

    def group(g, carry):
        b = lax.rem(g, NBUF)
        pltpu.make_async_copy(
            hb_hbm.at[idx_v.at[pl.ds(g * NB * K, NB * K)]], rows_v.at[b], sem
        ).wait()
        for nb in range(NB):
            i = g * NB + nb

            hb = [plsc.bitcast(hsb_v[i, pl.ds(c * 16, 16)], jnp.bfloat16)
                  for c in range(PCH)]

            def dot16(racc):
                ua, ub = plsc.unpack(racc, format=plsc.PackFormat.INTERLEAVED)
                return ua + ub

            def merge(a, bb, w2):
                p = lane ^ w2
                sa = a + _perm(a, p)
                sb = bb + _perm(bb, p)
                return jnp.where((lane & w2) == 0, sa, _perm(sb, p))

            acc = hb[0] * hb[0]
            for c in range(1, PCH):
                acc = acc + hb[c] * hb[c]
            s0v = dot16(acc)
            for w2 in (8, 4, 2, 1):
                s0v = s0v + _perm(s0v, lane ^ w2)

            quads = []
            for q in range(K // 4):
                sub = []
                for k4 in range(4):
                    k = nb * K + q * 4 + k4
                    a = plsc.bitcast(rows_v[b, k, pl.ds(0, 16)], jnp.bfloat16) * hb[0]
                    for c in range(1, PCH):
                        a = a + plsc.bitcast(
                            rows_v[b, k, pl.ds(c * 16, 16)], jnp.bfloat16) * hb[c]
                    sub.append(dot16(a))
                quads.append(merge(merge(sub[0], sub[1], 8),
                                   merge(sub[2], sub[3], 8), 4))
            v2 = [merge(quads[2 * j], quads[2 * j + 1], 2) for j in range(4)]
            sv0 = merge(v2[0], v2[1], 1)
            sv1 = merge(v2[2], v2[3], 1)

            e = jnp.exp(sv0 - s0v) + jnp.exp(sv1 - s0v)
            for w2 in (8, 4, 2, 1):
                e = e + _perm(e, lane ^ w2)
            wv = jnp.full((L,), 1.0, jnp.float32) / (1.0 + e)
            node = jnp.full((L,), base + i, jnp.int32)
            wv = jnp.where(node == 0, jnp.float32(0.0), wv)
            w_v[i, pl.ds(0, L)] = wv

        @pl.when(g + NBUF < ng)
        def _():
            pltpu.make_async_copy(
                hb_hbm.at[idx_v.at[pl.ds((g + NBUF) * NB * K, NB * K)]],
                rows_v.at[b], sem
            ).start()

        return carry

    lax.fori_loop(0, ng, group, 0)

    @pl.when(c == 0)
    def _():
        pltpu.sync_copy(w_v.at[pl.ds(0, NPW0)], w_hbm.at[pl.ds(base, NPW0)])

    @pl.when(c == 1)
    def _():
        pltpu.sync_copy(w_v.at[pl.ds(0, NPW1)], w_hbm.at[pl.ds(base, NPW1)])


@functools.cache
def _sc_attend():
    return pl.kernel(
        _sc_body,
        mesh=plsc.VectorSubcoreMesh(core_axis_name="c", subcore_axis_name="s"),
        out_type=jax.ShapeDtypeStruct((NPAD, L), jnp.float32),
        scratch_types=[
            pltpu.VMEM((NPW_MAX * K,), jnp.int32),
            pltpu.VMEM((NPW_MAX, D // 2), jnp.int32),
            pltpu.VMEM((NPW_MAX, L), jnp.float32),
            pltpu.VMEM((NBUF, NB * K, D // 2), jnp.int32),
            pltpu.SemaphoreType.DMA,
        ],
        compiler_params=pltpu.CompilerParams(
            needs_layout_passes=False, use_tc_tiling_on_sc=False),
    )


def kernel(x, a2a, W, b):
    x_pad = jnp.zeros((NPAD, D), jnp.float32).at[:N].set(x)
    a2a_pad = jnp.zeros((NPAD, K), jnp.int32).at[:N].set(a2a).reshape(NPAD * K)
    h, hb = _linear(x_pad, W.T, b[None, :])
    hb32 = lax.bitcast_convert_type(hb.reshape(NPAD, D // 2, 2), jnp.int32)
    w = _sc_attend()(hb32, a2a_pad)
    out = _scale(h, w)
    return out[:N]


# speedup vs baseline: 1.2397x; 1.2397x over previous
import functools

import jax
import jax.numpy as jnp
from jax import lax
from jax.experimental import pallas as pl
from jax.experimental.pallas import tpu as pltpu
from jax.experimental.pallas import tpu_sc as plsc

N, K, D = 10000, 32, 128
NPAD = 10240
NW = 32
NBUF = 2
NB = 4
L = 16
PCH = D // 32

NPW0 = 464
NPW1 = 640 - NPW0
NPW_MAX = max(NPW0, NPW1)

TC_BLK = 512


def _linear_body(x_ref, wt_ref, b_ref, h_ref, hb_ref):
    h = jnp.dot(x_ref[...], wt_ref[...], preferred_element_type=jnp.float32)
    h = h + b_ref[...]
    row = lax.broadcasted_iota(jnp.int32, h.shape, 0) + pl.program_id(0) * TC_BLK
    h = jnp.where(row == 0, jnp.float32(-9e15), h)
    h_ref[...] = h
    hb_ref[...] = h.astype(jnp.bfloat16)


def _linear(x_pad, Wt, b2):
    return pl.pallas_call(
        _linear_body,
        grid=(NPAD // TC_BLK,),
        in_specs=[
            pl.BlockSpec((TC_BLK, D), lambda i: (i, 0)),
            pl.BlockSpec((D, D), lambda i: (0, 0)),
            pl.BlockSpec((1, D), lambda i: (0, 0)),
        ],
        out_specs=[
            pl.BlockSpec((TC_BLK, D), lambda i: (i, 0)),
            pl.BlockSpec((TC_BLK, D), lambda i: (i, 0)),
        ],
        out_shape=[
            jax.ShapeDtypeStruct((NPAD, D), jnp.float32),
            jax.ShapeDtypeStruct((NPAD, D), jnp.bfloat16),
        ],
    )(x_pad, Wt, b2)


def _scale_body(h_ref, w_ref, o_ref):
    o_ref[...] = h_ref[...] * w_ref[:, 0:1]


def _scale(h, w):
    return pl.pallas_call(
        _scale_body,
        grid=(NPAD // TC_BLK,),
        in_specs=[
            pl.BlockSpec((TC_BLK, D), lambda i: (i, 0)),
            pl.BlockSpec((TC_BLK, L), lambda i: (i, 0)),
        ],
        out_specs=pl.BlockSpec((TC_BLK, D), lambda i: (i, 0)),
        out_shape=jax.ShapeDtypeStruct((NPAD, D), jnp.float32),
    )(h, w)


_GATHER_DNUMS = lax.GatherDimensionNumbers(
    offset_dims=(), collapsed_slice_dims=(0,), start_index_map=(0,))


def _perm(x, idx):
    return lax.gather(
        x, idx[:, None], _GATHER_DNUMS, (1,),
        unique_indices=True, indices_are_sorted=False,
        mode=lax.GatherScatterMode.PROMISE_IN_BOUNDS)


def _sc_body(hb_hbm, a2a_hbm, w_hbm, idx_v, hsb_v, w_v, rows_v, sem):
    c = lax.axis_index("c")
    s = lax.axis_index("s")
    base = s * (NPW0 + NPW1) + c * NPW0
    npw = jnp.where(c == 0, NPW0, NPW1)
    ng = npw // NB

    @pl.when(c == 0)
    def _():
        pltpu.sync_copy(a2a_hbm.at[pl.ds(base * K, NPW0 * K)],
                        idx_v.at[pl.ds(0, NPW0 * K)])
        pltpu.sync_copy(hb_hbm.at[pl.ds(base, NPW0)], hsb_v.at[pl.ds(0, NPW0)])

    @pl.when(c == 1)
    def _():
        pltpu.sync_copy(a2a_hbm.at[pl.ds(base * K, NPW1 * K)],
                        idx_v.at[pl.ds(0, NPW1 * K)])
        pltpu.sync_copy(hb_hbm.at[pl.ds(base, NPW1)], hsb_v.at[pl.ds(0, NPW1)])

    lane = lax.iota(jnp.int32, L)

    for b in range(NBUF):
        pltpu.make_async_copy(
            hb_hbm.at[idx_v.at[pl.ds(b * NB * K, NB * K)]], rows_v.at[b], sem
        ).start()

    def group(g, carry):
        b = lax.rem(g, NBUF)
        pltpu.make_async_copy(
            hb_hbm.at[idx_v.at[pl.ds(g * NB * K, NB * K)]], rows_v.at[b], sem
        ).wait()
        for nb in range(NB):
            i = g * NB + nb

            hb = [plsc.bitcast(hsb_v[i, pl.ds(c * 16, 16)], jnp.bfloat16)
                  for c in range(PCH)]

            def dot16(racc):
                ua, ub = plsc.unpack(racc, format=plsc.PackFormat.INTERLEAVED)
                return ua + ub

            def merge(a, bb, w2):
                p = lane ^ w2
                sa = a + _perm(a, p)
                sb = bb + _perm(bb, p)
                return jnp.where((lane & w2) == 0, sa, _perm(sb, p))

            acc = hb[0] * hb[0]
            for c in range(1, PCH):
                acc = acc + hb[c] * hb[c]
            s0v = dot16(acc)
            for w2 in (8, 4, 2, 1):
                s0v = s0v + _perm(s0v, lane ^ w2)

            quads = []
            for q in range(K // 4):
                sub = []
                for k4 in range(4):
                    k = nb * K + q * 4 + k4
                    a = plsc.bitcast(rows_v[b, k, pl.ds(0, 16)], jnp.bfloat16) * hb[0]
                    for c in range(1, PCH):
                        a = a + plsc.bitcast(
                            rows_v[b, k, pl.ds(c * 16, 16)], jnp.bfloat16) * hb[c]
                    sub.append(dot16(a))
                quads.append(merge(merge(sub[0], sub[1], 8),
                                   merge(sub[2], sub[3], 8), 4))
            v2 = [merge(quads[2 * j], quads[2 * j + 1], 2) for j in range(4)]
            sv0 = merge(v2[0], v2[1], 1)
            sv1 = merge(v2[2], v2[3], 1)

            e = jnp.exp(sv0 - s0v) + jnp.exp(sv1 - s0v)
            for w2 in (8, 4, 2, 1):
                e = e + _perm(e, lane ^ w2)
            wv = jnp.full((L,), 1.0, jnp.float32) / (1.0 + e)
            node = jnp.full((L,), base + i, jnp.int32)
            wv = jnp.where(node == 0, jnp.float32(0.0), wv)
            w_v[i, pl.ds(0, L)] = wv

        @pl.when(g + NBUF < ng)
        def _():
            pltpu.make_async_copy(
                hb_hbm.at[idx_v.at[pl.ds((g + NBUF) * NB * K, NB * K)]],
                rows_v.at[b], sem
            ).start()

        return carry

    lax.fori_loop(0, ng, group, 0)

    @pl.when(c == 0)
    def _():
        pltpu.sync_copy(w_v.at[pl.ds(0, NPW0)], w_hbm.at[pl.ds(base, NPW0)])

    @pl.when(c == 1)
    def _():
        pltpu.sync_copy(w_v.at[pl.ds(0, NPW1)], w_hbm.at[pl.ds(base, NPW1)])


@functools.cache
def _sc_attend():
    return pl.kernel(
        _sc_body,
        mesh=plsc.VectorSubcoreMesh(core_axis_name="c", subcore_axis_name="s"),
        out_type=jax.ShapeDtypeStruct((NPAD, L), jnp.float32),
        scratch_types=[
            pltpu.VMEM((NPW_MAX * K,), jnp.int32),
            pltpu.VMEM((NPW_MAX, D // 2), jnp.int32),
            pltpu.VMEM((NPW_MAX, L), jnp.float32),
            pltpu.VMEM((NBUF, NB * K, D // 2), jnp.int32),
            pltpu.SemaphoreType.DMA,
        ],
        compiler_params=pltpu.CompilerParams(
            needs_layout_passes=False, use_tc_tiling_on_sc=False),
    )


def kernel(x, a2a, W, b):
    x_pad = jnp.zeros((NPAD, D), jnp.float32).at[:N].set(x)
    a2a_pad = jnp.zeros((NPAD, K), jnp.int32).at[:N].set(a2a).reshape(NPAD * K)
    h, hb = _linear(x_pad, W.T, b[None, :])
    hb32 = lax.bitcast_convert_type(hb.reshape(NPAD, D // 2, 2), jnp.int32)
    w = _sc_attend()(hb32, a2a_pad)
    out = _scale(h, w)
    return out[:N]


# speedup vs baseline: 2.4296x; 1.9599x over previous
import functools

import jax
import jax.numpy as jnp
from jax import lax
from jax.experimental import pallas as pl
from jax.experimental.pallas import tpu as pltpu
from jax.experimental.pallas import tpu_sc as plsc

N, K, D = 10000, 32, 128
NBUF = 2
NB = 4
L = 16
PCH = D // 32

NPW0, NPW1 = 464, 176
NPW0L, NPW1L = 292, 108
NPW_MAX = NPW0

TC_BLK = 1000


def _linear_body(x_ref, wt_ref, b_ref, h_ref, hb_ref):
    h = jnp.dot(x_ref[...], wt_ref[...], preferred_element_type=jnp.float32)
    h = h + b_ref[...]
    row = lax.broadcasted_iota(jnp.int32, h.shape, 0) + pl.program_id(0) * TC_BLK
    h = jnp.where(row == 0, jnp.float32(-9e15), h)
    h_ref[...] = h
    u = lax.bitcast_convert_type(h.astype(jnp.bfloat16), jnp.uint16)
    lo = u[:, : D // 2].astype(jnp.uint32)
    hi = u[:, D // 2 :].astype(jnp.uint32)
    hb_ref[...] = lax.bitcast_convert_type(lo | (hi << 16), jnp.int32)


def _linear(x, Wt, b2):
    return pl.pallas_call(
        _linear_body,
        grid=(N // TC_BLK,),
        in_specs=[
            pl.BlockSpec((TC_BLK, D), lambda i: (i, 0)),
            pl.BlockSpec((D, D), lambda i: (0, 0)),
            pl.BlockSpec((1, D), lambda i: (0, 0)),
        ],
        out_specs=[
            pl.BlockSpec((TC_BLK, D), lambda i: (i, 0)),
            pl.BlockSpec((TC_BLK, D // 2), lambda i: (i, 0)),
        ],
        out_shape=[
            jax.ShapeDtypeStruct((N, D), jnp.float32),
            jax.ShapeDtypeStruct((N, D // 2), jnp.int32),
        ],
    )(x, Wt, b2)


def _scale_body(h_ref, w_ref, o_ref):
    o_ref[...] = h_ref[...] * w_ref[:, 0:1]


def _scale(h, w):
    return pl.pallas_call(
        _scale_body,
        grid=(N // TC_BLK,),
        in_specs=[
            pl.BlockSpec((TC_BLK, D), lambda i: (i, 0)),
            pl.BlockSpec((TC_BLK, L), lambda i: (i, 0)),
        ],
        out_specs=pl.BlockSpec((TC_BLK, D), lambda i: (i, 0)),
        out_shape=jax.ShapeDtypeStruct((N, D), jnp.float32),
    )(h, w)


_GATHER_DNUMS = lax.GatherDimensionNumbers(
    offset_dims=(), collapsed_slice_dims=(0,), start_index_map=(0,))


def _perm(x, idx):
    return lax.gather(
        x, idx[:, None], _GATHER_DNUMS, (1,),
        unique_indices=True, indices_are_sorted=False,
        mode=lax.GatherScatterMode.PROMISE_IN_BOUNDS)


def _sc_body(hb_hbm, a2a_hbm, w_hbm, idx_v, hsb_v, w_v, rows_v, sem):
    c = lax.axis_index("c")
    s = lax.axis_index("s")
    last = s == 15
    base = jnp.where(last, 15 * 640 + c * NPW0L, s * 640 + c * NPW0)
    npw = jnp.where(last,
                    jnp.where(c == 0, NPW0L, NPW1L),
                    jnp.where(c == 0, NPW0, NPW1))
    ng = npw // NB

    def stage(n_nodes):
        pltpu.sync_copy(a2a_hbm.at[pl.ds(base * K, n_nodes * K)],
                        idx_v.at[pl.ds(0, n_nodes * K)])
        pltpu.sync_copy(hb_hbm.at[pl.ds(base, n_nodes)],
                        hsb_v.at[pl.ds(0, n_nodes)])

    def unstage(n_nodes):
        pltpu.sync_copy(w_v.at[pl.ds(0, n_nodes)],
                        w_hbm.at[pl.ds(base, n_nodes)])

    for cc, nn, nl in ((0, NPW0, NPW0L), (1, NPW1, NPW1L)):
        @pl.when((c == cc) & ~last)
        def _(nn=nn):
            stage(nn)

        @pl.when((c == cc) & last)
        def _(nl=nl):
            stage(nl)

    lane = lax.iota(jnp.int32, L)

    for b in range(NBUF):
        pltpu.make_async_copy(
            hb_hbm.at[idx_v.at[pl.ds(b * NB * K, NB * K)]], rows_v.at[b], sem
        ).start()

    def group(g, carry):
        b = lax.rem(g, NBUF)
        pltpu.make_async_copy(
            hb_hbm.at[idx_v.at[pl.ds(g * NB * K, NB * K)]], rows_v.at[b], sem
        ).wait()
        for nb in range(NB):
            i = g * NB + nb

            hb = [plsc.bitcast(hsb_v[i, pl.ds(ch * 16, 16)], jnp.bfloat16)
                  for ch in range(PCH)]

            def dot16(racc):
                ua, ub = plsc.unpack(racc, format=plsc.PackFormat.INTERLEAVED)
                return ua + ub

            def merge(a, bb, w2):
                p = lane ^ w2
                sa = a + _perm(a, p)
                sb = bb + _perm(bb, p)
                return jnp.where((lane & w2) == 0, sa, _perm(sb, p))

            acc = hb[0] * hb[0]
            for ch in range(1, PCH):
                acc = acc + hb[ch] * hb[ch]
            s0v = dot16(acc)
            for w2 in (8, 4, 2, 1):
                s0v = s0v + _perm(s0v, lane ^ w2)

            quads = []
            for q in range(K // 4):
                sub = []
                for k4 in range(4):
                    k = nb * K + q * 4 + k4
                    a = plsc.bitcast(rows_v[b, k, pl.ds(0, 16)], jnp.bfloat16) * hb[0]
                    for ch in range(1, PCH):
                        a = a + plsc.bitcast(
                            rows_v[b, k, pl.ds(ch * 16, 16)], jnp.bfloat16) * hb[ch]
                    sub.append(dot16(a))
                quads.append(merge(merge(sub[0], sub[1], 8),
                                   merge(sub[2], sub[3], 8), 4))
            v2 = [merge(quads[2 * j], quads[2 * j + 1], 2) for j in range(4)]
            sv0 = merge(v2[0], v2[1], 1)
            sv1 = merge(v2[2], v2[3], 1)

            e = jnp.exp(sv0 - s0v) + jnp.exp(sv1 - s0v)
            for w2 in (8, 4, 2, 1):
                e = e + _perm(e, lane ^ w2)
            wv = jnp.full((L,), 1.0, jnp.float32) / (1.0 + e)
            node = jnp.full((L,), base + i, jnp.int32)
            wv = jnp.where(node == 0, jnp.float32(0.0), wv)
            w_v[i, pl.ds(0, L)] = wv

        @pl.when(g + NBUF < ng)
        def _():
            pltpu.make_async_copy(
                hb_hbm.at[idx_v.at[pl.ds((g + NBUF) * NB * K, NB * K)]],
                rows_v.at[b], sem
            ).start()

        return carry

    lax.fori_loop(0, ng, group, 0)

    for cc, nn, nl in ((0, NPW0, NPW0L), (1, NPW1, NPW1L)):
        @pl.when((c == cc) & ~last)
        def _(nn=nn):
            unstage(nn)

        @pl.when((c == cc) & last)
        def _(nl=nl):
            unstage(nl)


@functools.cache
def _sc_attend():
    return pl.kernel(
        _sc_body,
        mesh=plsc.VectorSubcoreMesh(core_axis_name="c", subcore_axis_name="s"),
        out_type=jax.ShapeDtypeStruct((N, L), jnp.float32),
        scratch_types=[
            pltpu.VMEM((NPW_MAX * K,), jnp.int32),
            pltpu.VMEM((NPW_MAX, D // 2), jnp.int32),
            pltpu.VMEM((NPW_MAX, L), jnp.float32),
            pltpu.VMEM((NBUF, NB * K, D // 2), jnp.int32),
            pltpu.SemaphoreType.DMA,
        ],
        compiler_params=pltpu.CompilerParams(
            needs_layout_passes=False, use_tc_tiling_on_sc=False),
    )


def kernel(x, a2a, W, b):
    h, hb32 = _linear(x, W.T, b[None, :])
    w = _sc_attend()(hb32, a2a.reshape(N * K))
    return _scale(h, w)


# speedup vs baseline: 3.5885x; 1.4770x over previous
import functools

import jax
import jax.numpy as jnp
from jax import lax
from jax.experimental import pallas as pl
from jax.experimental.pallas import tpu as pltpu
from jax.experimental.pallas import tpu_sc as plsc

N, K, D = 10000, 32, 128
NBUF = 4
NB = 4
L = 16
PCH = D // 32

NPW0, NPW1 = 320, 320
NPW0L, NPW1L = 200, 200
NPW_MAX = NPW0

TC_BLK = 2000


def _linear_body(x_ref, wt_ref, b_ref, h_ref, hb_ref):
    h = jnp.dot(x_ref[...], wt_ref[...], preferred_element_type=jnp.float32)
    h = h + b_ref[...]
    row = lax.broadcasted_iota(jnp.int32, h.shape, 0) + pl.program_id(0) * TC_BLK
    h = jnp.where(row == 0, jnp.float32(-9e15), h)
    h_ref[...] = h
    u = lax.bitcast_convert_type(h.astype(jnp.bfloat16), jnp.uint16)
    lo = u[:, : D // 2].astype(jnp.uint32)
    hi = u[:, D // 2 :].astype(jnp.uint32)
    hb_ref[...] = lax.bitcast_convert_type(lo | (hi << 16), jnp.int32)


def _linear(x, Wt, b2):
    return pl.pallas_call(
        _linear_body,
        grid=(N // TC_BLK,),
        in_specs=[
            pl.BlockSpec((TC_BLK, D), lambda i: (i, 0)),
            pl.BlockSpec((D, D), lambda i: (0, 0)),
            pl.BlockSpec((1, D), lambda i: (0, 0)),
        ],
        out_specs=[
            pl.BlockSpec((TC_BLK, D), lambda i: (i, 0)),
            pl.BlockSpec((TC_BLK, D // 2), lambda i: (i, 0)),
        ],
        out_shape=[
            jax.ShapeDtypeStruct((N, D), jnp.float32),
            jax.ShapeDtypeStruct((N, D // 2), jnp.int32),
        ],
    )(x, Wt, b2)


def _scale_body(h_ref, w_ref, o_ref):
    o_ref[...] = h_ref[...] * w_ref[:, 0:1]


def _scale(h, w):
    return pl.pallas_call(
        _scale_body,
        grid=(N // TC_BLK,),
        in_specs=[
            pl.BlockSpec((TC_BLK, D), lambda i: (i, 0)),
            pl.BlockSpec((TC_BLK, L), lambda i: (i, 0)),
        ],
        out_specs=pl.BlockSpec((TC_BLK, D), lambda i: (i, 0)),
        out_shape=jax.ShapeDtypeStruct((N, D), jnp.float32),
    )(h, w)


_GATHER_DNUMS = lax.GatherDimensionNumbers(
    offset_dims=(), collapsed_slice_dims=(0,), start_index_map=(0,))


def _perm(x, idx):
    return lax.gather(
        x, idx[:, None], _GATHER_DNUMS, (1,),
        unique_indices=True, indices_are_sorted=False,
        mode=lax.GatherScatterMode.PROMISE_IN_BOUNDS)


def _sc_body(hb_hbm, a2a_hbm, w_hbm, idx_v, hsb_v, w_v, rows_v, sem):
    c = lax.axis_index("c")
    s = lax.axis_index("s")
    last = s == 15
    base = jnp.where(last, 15 * 640 + c * NPW0L, s * 640 + c * NPW0)
    npw = jnp.where(last,
                    jnp.where(c == 0, NPW0L, NPW1L),
                    jnp.where(c == 0, NPW0, NPW1))
    ng = npw // NB

    def stage(n_nodes):
        pltpu.sync_copy(a2a_hbm.at[pl.ds(base * K, n_nodes * K)],
                        idx_v.at[pl.ds(0, n_nodes * K)])
        pltpu.sync_copy(hb_hbm.at[pl.ds(base, n_nodes)],
                        hsb_v.at[pl.ds(0, n_nodes)])

    def unstage(n_nodes):
        pltpu.sync_copy(w_v.at[pl.ds(0, n_nodes)],
                        w_hbm.at[pl.ds(base, n_nodes)])

    for cc, nn, nl in ((0, NPW0, NPW0L), (1, NPW1, NPW1L)):
        @pl.when((c == cc) & ~last)
        def _(nn=nn):
            stage(nn)

        @pl.when((c == cc) & last)
        def _(nl=nl):
            stage(nl)

    lane = lax.iota(jnp.int32, L)

    for b in range(NBUF):
        pltpu.make_async_copy(
            hb_hbm.at[idx_v.at[pl.ds(b * NB * K, NB * K)]], rows_v.at[b], sem
        ).start()

    def group(g, carry):
        b = lax.rem(g, NBUF)
        pltpu.make_async_copy(
            hb_hbm.at[idx_v.at[pl.ds(g * NB * K, NB * K)]], rows_v.at[b], sem
        ).wait()
        for nb in range(NB):
            i = g * NB + nb

            hb = [plsc.bitcast(hsb_v[i, pl.ds(ch * 16, 16)], jnp.bfloat16)
                  for ch in range(PCH)]

            def dot16(racc):
                ua, ub = plsc.unpack(racc, format=plsc.PackFormat.INTERLEAVED)
                return ua + ub

            def merge(a, bb, w2):
                p = lane ^ w2
                sa = a + _perm(a, p)
                sb = bb + _perm(bb, p)
                return jnp.where((lane & w2) == 0, sa, _perm(sb, p))

            acc = hb[0] * hb[0]
            for ch in range(1, PCH):
                acc = acc + hb[ch] * hb[ch]
            s0v = dot16(acc)
            for w2 in (8, 4, 2, 1):
                s0v = s0v + _perm(s0v, lane ^ w2)

            quads = []
            for q in range(K // 4):
                sub = []
                for k4 in range(4):
                    k = nb * K + q * 4 + k4
                    a = plsc.bitcast(rows_v[b, k, pl.ds(0, 16)], jnp.bfloat16) * hb[0]
                    for ch in range(1, PCH):
                        a = a + plsc.bitcast(
                            rows_v[b, k, pl.ds(ch * 16, 16)], jnp.bfloat16) * hb[ch]
                    sub.append(dot16(a))
                quads.append(merge(merge(sub[0], sub[1], 8),
                                   merge(sub[2], sub[3], 8), 4))
            v2 = [merge(quads[2 * j], quads[2 * j + 1], 2) for j in range(4)]
            sv0 = merge(v2[0], v2[1], 1)
            sv1 = merge(v2[2], v2[3], 1)

            e = jnp.exp(sv0 - s0v) + jnp.exp(sv1 - s0v)
            for w2 in (8, 4, 2, 1):
                e = e + _perm(e, lane ^ w2)
            wv = jnp.full((L,), 1.0, jnp.float32) / (1.0 + e)
            node = jnp.full((L,), base + i, jnp.int32)
            wv = jnp.where(node == 0, jnp.float32(0.0), wv)
            w_v[i, pl.ds(0, L)] = wv

        @pl.when(g + NBUF < ng)
        def _():
            pltpu.make_async_copy(
                hb_hbm.at[idx_v.at[pl.ds((g + NBUF) * NB * K, NB * K)]],
                rows_v.at[b], sem
            ).start()

        return carry

    lax.fori_loop(0, ng, group, 0)

    for cc, nn, nl in ((0, NPW0, NPW0L), (1, NPW1, NPW1L)):
        @pl.when((c == cc) & ~last)
        def _(nn=nn):
            unstage(nn)

        @pl.when((c == cc) & last)
        def _(nl=nl):
            unstage(nl)


@functools.cache
def _sc_attend():
    return pl.kernel(
        _sc_body,
        mesh=plsc.VectorSubcoreMesh(core_axis_name="c", subcore_axis_name="s"),
        out_type=jax.ShapeDtypeStruct((N, L), jnp.float32),
        scratch_types=[
            pltpu.VMEM((NPW_MAX * K,), jnp.int32),
            pltpu.VMEM((NPW_MAX, D // 2), jnp.int32),
            pltpu.VMEM((NPW_MAX, L), jnp.float32),
            pltpu.VMEM((NBUF, NB * K, D // 2), jnp.int32),
            pltpu.SemaphoreType.DMA,
        ],
        compiler_params=pltpu.CompilerParams(
            needs_layout_passes=False, use_tc_tiling_on_sc=False),
    )


def kernel(x, a2a, W, b):
    h, hb32 = _linear(x, W.T, b[None, :])
    w = _sc_attend()(hb32, a2a.reshape(N * K))
    return _scale(h, w)


# speedup vs baseline: 3.6464x; 1.0161x over previous
import functools

import jax
import jax.numpy as jnp
from jax import lax
from jax.experimental import pallas as pl
from jax.experimental.pallas import tpu as pltpu
from jax.experimental.pallas import tpu_sc as plsc

N, K, D = 10000, 32, 128
NBUF = 4
NB = 4
L = 16
PCH = D // 32

NPW0, NPW1 = 320, 320
NPW0L, NPW1L = 200, 200
NPW_MAX = NPW0

TC_BLK = 2000


def _linear_body(x_ref, w_ref, b_ref, h_ref, hb_ref):
    h = lax.dot_general(x_ref[...], w_ref[...], (((1,), (1,)), ((), ())),
                        preferred_element_type=jnp.float32)
    h = h + b_ref[...]
    row = lax.broadcasted_iota(jnp.int32, h.shape, 0) + pl.program_id(0) * TC_BLK
    h = jnp.where(row == 0, jnp.float32(-9e15), h)
    h_ref[...] = h
    u = lax.bitcast_convert_type(h.astype(jnp.bfloat16), jnp.uint16)
    lo = u[:, : D // 2].astype(jnp.uint32)
    hi = u[:, D // 2 :].astype(jnp.uint32)
    hb_ref[...] = lax.bitcast_convert_type(lo | (hi << 16), jnp.int32)


def _linear(x, W, b2):
    return pl.pallas_call(
        _linear_body,
        grid=(N // TC_BLK,),
        in_specs=[
            pl.BlockSpec((TC_BLK, D), lambda i: (i, 0)),
            pl.BlockSpec((D, D), lambda i: (0, 0)),
            pl.BlockSpec((1, D), lambda i: (0, 0)),
        ],
        out_specs=[
            pl.BlockSpec((TC_BLK, D), lambda i: (i, 0)),
            pl.BlockSpec((TC_BLK, D // 2), lambda i: (i, 0)),
        ],
        out_shape=[
            jax.ShapeDtypeStruct((N, D), jnp.float32),
            jax.ShapeDtypeStruct((N, D // 2), jnp.int32),
        ],
    )(x, W, b2)


def _scale_body(h_ref, w_ref, o_ref):
    o_ref[...] = h_ref[...] * w_ref[:, 0:1]


def _scale(h, w):
    return pl.pallas_call(
        _scale_body,
        grid=(N // TC_BLK,),
        in_specs=[
            pl.BlockSpec((TC_BLK, D), lambda i: (i, 0)),
            pl.BlockSpec((TC_BLK, L), lambda i: (i, 0)),
        ],
        out_specs=pl.BlockSpec((TC_BLK, D), lambda i: (i, 0)),
        out_shape=jax.ShapeDtypeStruct((N, D), jnp.float32),
    )(h, w)


_GATHER_DNUMS = lax.GatherDimensionNumbers(
    offset_dims=(), collapsed_slice_dims=(0,), start_index_map=(0,))


def _perm(x, idx):
    return lax.gather(
        x, idx[:, None], _GATHER_DNUMS, (1,),
        unique_indices=True, indices_are_sorted=False,
        mode=lax.GatherScatterMode.PROMISE_IN_BOUNDS)


def _sc_body(hb_hbm, a2a_hbm, w_hbm, idx_v, hsb_v, w_v, rows_v, sem):
    c = lax.axis_index("c")
    s = lax.axis_index("s")
    last = s == 15
    base = jnp.where(last, 15 * 640 + c * NPW0L, s * 640 + c * NPW0)
    npw = jnp.where(last,
                    jnp.where(c == 0, NPW0L, NPW1L),
                    jnp.where(c == 0, NPW0, NPW1))
    ng = npw // NB

    def stage(n_nodes):
        pltpu.sync_copy(a2a_hbm.at[pl.ds(base * K, n_nodes * K)],
                        idx_v.at[pl.ds(0, n_nodes * K)])
        pltpu.sync_copy(hb_hbm.at[pl.ds(base, n_nodes)],
                        hsb_v.at[pl.ds(0, n_nodes)])

    def unstage(n_nodes):
        pltpu.sync_copy(w_v.at[pl.ds(0, n_nodes)],
                        w_hbm.at[pl.ds(base, n_nodes)])

    for cc, nn, nl in ((0, NPW0, NPW0L), (1, NPW1, NPW1L)):
        @pl.when((c == cc) & ~last)
        def _(nn=nn):
            stage(nn)

        @pl.when((c == cc) & last)
        def _(nl=nl):
            stage(nl)

    lane = lax.iota(jnp.int32, L)

    for b in range(NBUF):
        pltpu.make_async_copy(
            hb_hbm.at[idx_v.at[pl.ds(b * NB * K, NB * K)]], rows_v.at[b], sem
        ).start()

    def group(g, carry):
        b = lax.rem(g, NBUF)
        pltpu.make_async_copy(
            hb_hbm.at[idx_v.at[pl.ds(g * NB * K, NB * K)]], rows_v.at[b], sem
        ).wait()
        for nb in range(NB):
            i = g * NB + nb

            hb = [plsc.bitcast(hsb_v[i, pl.ds(ch * 16, 16)], jnp.bfloat16)
                  for ch in range(PCH)]

            def dot16(racc):
                ua, ub = plsc.unpack(racc, format=plsc.PackFormat.INTERLEAVED)
                return ua + ub

            def merge(a, bb, w2):
                p = lane ^ w2
                sa = a + _perm(a, p)
                sb = bb + _perm(bb, p)
                return jnp.where((lane & w2) == 0, sa, _perm(sb, p))

            acc = hb[0] * hb[0]
            for ch in range(1, PCH):
                acc = acc + hb[ch] * hb[ch]
            s0v = dot16(acc)
            for w2 in (8, 4, 2, 1):
                s0v = s0v + _perm(s0v, lane ^ w2)

            quads = []
            for q in range(K // 4):
                sub = []
                for k4 in range(4):
                    k = nb * K + q * 4 + k4
                    a = plsc.bitcast(rows_v[b, k, pl.ds(0, 16)], jnp.bfloat16) * hb[0]
                    for ch in range(1, PCH):
                        a = a + plsc.bitcast(
                            rows_v[b, k, pl.ds(ch * 16, 16)], jnp.bfloat16) * hb[ch]
                    sub.append(dot16(a))
                quads.append(merge(merge(sub[0], sub[1], 8),
                                   merge(sub[2], sub[3], 8), 4))
            v2 = [merge(quads[2 * j], quads[2 * j + 1], 2) for j in range(4)]
            sv0 = merge(v2[0], v2[1], 1)
            sv1 = merge(v2[2], v2[3], 1)

            e = jnp.exp(sv0 - s0v) + jnp.exp(sv1 - s0v)
            for w2 in (8, 4, 2, 1):
                e = e + _perm(e, lane ^ w2)
            wv = jnp.full((L,), 1.0, jnp.float32) / (1.0 + e)
            node = jnp.full((L,), base + i, jnp.int32)
            wv = jnp.where(node == 0, jnp.float32(0.0), wv)
            w_v[i, pl.ds(0, L)] = wv

        @pl.when(g + NBUF < ng)
        def _():
            pltpu.make_async_copy(
                hb_hbm.at[idx_v.at[pl.ds((g + NBUF) * NB * K, NB * K)]],
                rows_v.at[b], sem
            ).start()

        return carry

    lax.fori_loop(0, ng, group, 0)

    for cc, nn, nl in ((0, NPW0, NPW0L), (1, NPW1, NPW1L)):
        @pl.when((c == cc) & ~last)
        def _(nn=nn):
            unstage(nn)

        @pl.when((c == cc) & last)
        def _(nl=nl):
            unstage(nl)


@functools.cache
def _sc_attend():
    return pl.kernel(
        _sc_body,
        mesh=plsc.VectorSubcoreMesh(core_axis_name="c", subcore_axis_name="s"),
        out_type=jax.ShapeDtypeStruct((N, L), jnp.float32),
        scratch_types=[
            pltpu.VMEM((NPW_MAX * K,), jnp.int32),
            pltpu.VMEM((NPW_MAX, D // 2), jnp.int32),
            pltpu.VMEM((NPW_MAX, L), jnp.float32),
            pltpu.VMEM((NBUF, NB * K, D // 2), jnp.int32),
            pltpu.SemaphoreType.DMA,
        ],
        compiler_params=pltpu.CompilerParams(
            needs_layout_passes=False, use_tc_tiling_on_sc=False),
    )


def kernel(x, a2a, W, b):
    h, hb32 = _linear(x, W, b[None, :])
    w = _sc_attend()(hb32, a2a.reshape(N * K))
    return _scale(h, w)


# speedup vs baseline: 3.6509x; 1.0012x over previous
import functools

import jax
import jax.numpy as jnp
from jax import lax
from jax.experimental import pallas as pl
from jax.experimental.pallas import tpu as pltpu
from jax.experimental.pallas import tpu_sc as plsc

N, K, D = 10000, 32, 128
NBUF = 8
NB = 4
L = 16
PCH = D // 32

NPW0, NPW1 = 320, 320
NPW0L, NPW1L = 200, 200
NPW_MAX = NPW0

TC_BLK = 2000


def _linear_body(x_ref, w_ref, b_ref, h_ref, hb_ref):
    h = lax.dot_general(x_ref[...], w_ref[...], (((1,), (1,)), ((), ())),
                        preferred_element_type=jnp.float32)
    h = h + b_ref[...]
    row = lax.broadcasted_iota(jnp.int32, h.shape, 0) + pl.program_id(0) * TC_BLK
    h = jnp.where(row == 0, jnp.float32(-9e15), h)
    h_ref[...] = h
    u = lax.bitcast_convert_type(h.astype(jnp.bfloat16), jnp.uint16)
    lo = u[:, : D // 2].astype(jnp.uint32)
    hi = u[:, D // 2 :].astype(jnp.uint32)
    hb_ref[...] = lax.bitcast_convert_type(lo | (hi << 16), jnp.int32)


def _linear(x, W, b2):
    return pl.pallas_call(
        _linear_body,
        grid=(N // TC_BLK,),
        in_specs=[
            pl.BlockSpec((TC_BLK, D), lambda i: (i, 0)),
            pl.BlockSpec((D, D), lambda i: (0, 0)),
            pl.BlockSpec((1, D), lambda i: (0, 0)),
        ],
        out_specs=[
            pl.BlockSpec((TC_BLK, D), lambda i: (i, 0)),
            pl.BlockSpec((TC_BLK, D // 2), lambda i: (i, 0)),
        ],
        out_shape=[
            jax.ShapeDtypeStruct((N, D), jnp.float32),
            jax.ShapeDtypeStruct((N, D // 2), jnp.int32),
        ],
    )(x, W, b2)


def _scale_body(h_ref, w_ref, o_ref):
    o_ref[...] = h_ref[...] * w_ref[:, 0:1]


def _scale(h, w):
    return pl.pallas_call(
        _scale_body,
        grid=(N // TC_BLK,),
        in_specs=[
            pl.BlockSpec((TC_BLK, D), lambda i: (i, 0)),
            pl.BlockSpec((TC_BLK, L), lambda i: (i, 0)),
        ],
        out_specs=pl.BlockSpec((TC_BLK, D), lambda i: (i, 0)),
        out_shape=jax.ShapeDtypeStruct((N, D), jnp.float32),
    )(h, w)


_GATHER_DNUMS = lax.GatherDimensionNumbers(
    offset_dims=(), collapsed_slice_dims=(0,), start_index_map=(0,))


def _perm(x, idx):
    return lax.gather(
        x, idx[:, None], _GATHER_DNUMS, (1,),
        unique_indices=True, indices_are_sorted=False,
        mode=lax.GatherScatterMode.PROMISE_IN_BOUNDS)


def _sc_body(hb_hbm, a2a_hbm, w_hbm, idx_v, hsb_v, w_v, rows_v, sem):
    c = lax.axis_index("c")
    s = lax.axis_index("s")
    last = s == 15
    base = jnp.where(last, 15 * 640 + c * NPW0L, s * 640 + c * NPW0)
    npw = jnp.where(last,
                    jnp.where(c == 0, NPW0L, NPW1L),
                    jnp.where(c == 0, NPW0, NPW1))
    ng = npw // NB

    def stage(n_nodes):
        pltpu.sync_copy(a2a_hbm.at[pl.ds(base * K, n_nodes * K)],
                        idx_v.at[pl.ds(0, n_nodes * K)])
        pltpu.sync_copy(hb_hbm.at[pl.ds(base, n_nodes)],
                        hsb_v.at[pl.ds(0, n_nodes)])

    def unstage(n_nodes):
        pltpu.sync_copy(w_v.at[pl.ds(0, n_nodes)],
                        w_hbm.at[pl.ds(base, n_nodes)])

    for cc, nn, nl in ((0, NPW0, NPW0L), (1, NPW1, NPW1L)):
        @pl.when((c == cc) & ~last)
        def _(nn=nn):
            stage(nn)

        @pl.when((c == cc) & last)
        def _(nl=nl):
            stage(nl)

    lane = lax.iota(jnp.int32, L)

    for b in range(NBUF):
        pltpu.make_async_copy(
            hb_hbm.at[idx_v.at[pl.ds(b * NB * K, NB * K)]], rows_v.at[b], sem
        ).start()

    def group(g, carry):
        b = lax.rem(g, NBUF)
        pltpu.make_async_copy(
            hb_hbm.at[idx_v.at[pl.ds(g * NB * K, NB * K)]], rows_v.at[b], sem
        ).wait()
        for nb in range(NB):
            i = g * NB + nb

            hb = [plsc.bitcast(hsb_v[i, pl.ds(ch * 16, 16)], jnp.bfloat16)
                  for ch in range(PCH)]

            def dot16(racc):
                ua, ub = plsc.unpack(racc, format=plsc.PackFormat.INTERLEAVED)
                return ua + ub

            def merge(a, bb, w2):
                p = lane ^ w2
                sa = a + _perm(a, p)
                sb = bb + _perm(bb, p)
                return jnp.where((lane & w2) == 0, sa, _perm(sb, p))

            acc = hb[0] * hb[0]
            for ch in range(1, PCH):
                acc = acc + hb[ch] * hb[ch]
            s0v = dot16(acc)
            for w2 in (8, 4, 2, 1):
                s0v = s0v + _perm(s0v, lane ^ w2)

            quads = []
            for q in range(K // 4):
                sub = []
                for k4 in range(4):
                    k = nb * K + q * 4 + k4
                    a = plsc.bitcast(rows_v[b, k, pl.ds(0, 16)], jnp.bfloat16) * hb[0]
                    for ch in range(1, PCH):
                        a = a + plsc.bitcast(
                            rows_v[b, k, pl.ds(ch * 16, 16)], jnp.bfloat16) * hb[ch]
                    sub.append(dot16(a))
                quads.append(merge(merge(sub[0], sub[1], 8),
                                   merge(sub[2], sub[3], 8), 4))
            v2 = [merge(quads[2 * j], quads[2 * j + 1], 2) for j in range(4)]
            sv0 = merge(v2[0], v2[1], 1)
            sv1 = merge(v2[2], v2[3], 1)

            e = jnp.exp(sv0 - s0v) + jnp.exp(sv1 - s0v)
            for w2 in (8, 4, 2, 1):
                e = e + _perm(e, lane ^ w2)
            wv = jnp.full((L,), 1.0, jnp.float32) / (1.0 + e)
            node = jnp.full((L,), base + i, jnp.int32)
            wv = jnp.where(node == 0, jnp.float32(0.0), wv)
            w_v[i, pl.ds(0, L)] = wv

        @pl.when(g + NBUF < ng)
        def _():
            pltpu.make_async_copy(
                hb_hbm.at[idx_v.at[pl.ds((g + NBUF) * NB * K, NB * K)]],
                rows_v.at[b], sem
            ).start()

        return carry

    lax.fori_loop(0, ng, group, 0)

    for cc, nn, nl in ((0, NPW0, NPW0L), (1, NPW1, NPW1L)):
        @pl.when((c == cc) & ~last)
        def _(nn=nn):
            unstage(nn)

        @pl.when((c == cc) & last)
        def _(nl=nl):
            unstage(nl)


@functools.cache
def _sc_attend():
    return pl.kernel(
        _sc_body,
        mesh=plsc.VectorSubcoreMesh(core_axis_name="c", subcore_axis_name="s"),
        out_type=jax.ShapeDtypeStruct((N, L), jnp.float32),
        scratch_types=[
            pltpu.VMEM((NPW_MAX * K,), jnp.int32),
            pltpu.VMEM((NPW_MAX, D // 2), jnp.int32),
            pltpu.VMEM((NPW_MAX, L), jnp.float32),
            pltpu.VMEM((NBUF, NB * K, D // 2), jnp.int32),
            pltpu.SemaphoreType.DMA,
        ],
        compiler_params=pltpu.CompilerParams(
            needs_layout_passes=False, use_tc_tiling_on_sc=False),
    )


def kernel(x, a2a, W, b):
    h, hb32 = _linear(x, W, b[None, :])
    w = _sc_attend()(hb32, a2a.reshape(N * K))
    return _scale(h, w)
